# baseline (speedup 1.0000x reference)
import jax
import jax.numpy as jnp
from jax import lax
from jax.experimental import pallas as pl
from jax.experimental.pallas import tpu as pltpu
from jax.experimental.pallas import tpu_sc as plsc

D_MODEL = 64
SCALE = 8.0
NCORES = 2
NSUB = 16
NW = NCORES * NSUB
W = 512
NBUF = 3
B0 = 256


def _gather_rows(table, idx, n):
    per_w = n // NW
    nchunk = per_w // W

    mesh = plsc.VectorSubcoreMesh(core_axis_name="core",
                                  subcore_axis_name="subcore")

    @pl.kernel(out_type=jax.ShapeDtypeStruct((n, D_MODEL), table.dtype),
               mesh=mesh,
               scratch_types=[
                   pltpu.VMEM((per_w,), jnp.int32),
                   pltpu.VMEM((NBUF, W, D_MODEL), jnp.float32),
                   pltpu.SemaphoreType.DMA((NBUF,)),
                   pltpu.SemaphoreType.DMA((NBUF,)),
               ],
               compiler_params=pltpu.CompilerParams(use_tc_tiling_on_sc=False))
    def emb_kernel(table_hbm, idx_hbm, out_hbm, idx_v, rows_v, gsem, osem):
        wid = lax.axis_index("subcore") * NCORES + lax.axis_index("core")
        base = wid * per_w
        pltpu.sync_copy(idx_hbm.at[pl.ds(base, per_w)], idx_v)

        def gather(c):
            return pltpu.async_copy(
                table_hbm.at[idx_v.at[pl.ds(c * W, W)]],
                rows_v.at[c % NBUF], gsem.at[c % NBUF])

        ghandles = [gather(0), gather(1)]
        ohandles = [None] * NBUF
        for c in range(nchunk):
            bb = c % NBUF
            if c + 2 < nchunk:
                nb = (c + 2) % NBUF
                if ohandles[nb] is not None:
                    ohandles[nb].wait()
                ghandles.append(gather(c + 2))
            ghandles[c].wait()
            ohandles[bb] = pltpu.async_copy(
                rows_v.at[bb], out_hbm.at[pl.ds(base + c * W, W)],
                osem.at[bb])
        for h in ohandles:
            if h is not None:
                h.wait()

    return emb_kernel(table, idx)


def _transpose_scale(y128, b, l):
    nj = b // 2 // B0

    def body(y_ref, o_ref):
        val = y_ref[...]
        h = pl.program_id(2)
        half = jnp.where(h == 0, val[:, :D_MODEL], val[:, D_MODEL:])
        o_ref[...] = (half.T * SCALE)[None]

    return pl.pallas_call(
        body,
        grid=(l, nj, 2),
        in_specs=[pl.BlockSpec((B0, 128), lambda i, jj, h: (i * nj + jj, 0))],
        out_specs=pl.BlockSpec((1, D_MODEL, B0),
                               lambda i, jj, h: (i, 0, h * nj + jj)),
        out_shape=jax.ShapeDtypeStruct((l, D_MODEL, b), jnp.float32),
    )(y128)


def kernel(x, table):
    b, l = x.shape
    n = b * l
    idx = x.T.reshape(l, 2, b // 2).transpose(0, 2, 1).reshape(n)
    y = _gather_rows(table, idx, n)
    out_t = _transpose_scale(y.reshape(n // 2, 128), b, l)
    return out_t.transpose(2, 0, 1)


# speedup vs baseline: 2.0284x; 2.0284x over previous
import jax
import jax.numpy as jnp
from jax import lax
from jax.experimental import pallas as pl
from jax.experimental.pallas import tpu as pltpu
from jax.experimental.pallas import tpu_sc as plsc

D_MODEL = 64
SCALE = 8.0
NCORES = 2
NSUB = 16
NW = NCORES * NSUB
W = 256
NBUF = 3
V0 = 2048
B0 = 1024


def _prep_table(table_t):
    d, v = table_t.shape
    grid = (v + V0 - 1) // V0

    def body(t_ref, o_ref):
        o_ref[:, :D_MODEL] = t_ref[...].T

    return pl.pallas_call(
        body,
        grid=(grid,),
        in_specs=[pl.BlockSpec((d, V0), lambda i: (0, i))],
        out_specs=pl.BlockSpec((V0, 128), lambda i: (i, 0)),
        out_shape=jax.ShapeDtypeStruct((v, 128), jnp.float32),
    )(table_t)


def _gather_rows(table128, idx, n, b):
    per_w = n // NW
    nchunk = per_w // W
    half = b // 2

    mesh = plsc.VectorSubcoreMesh(core_axis_name="core",
                                  subcore_axis_name="subcore")

    @pl.kernel(out_type=jax.ShapeDtypeStruct((n // 2, 128), jnp.float32),
               mesh=mesh,
               scratch_types=[
                   pltpu.VMEM((per_w,), jnp.int32),
                   pltpu.VMEM((NBUF, W, 128), jnp.float32),
                   pltpu.SemaphoreType.DMA((NBUF,)),
                   pltpu.SemaphoreType.DMA((NBUF,)),
               ],
               compiler_params=pltpu.CompilerParams(use_tc_tiling_on_sc=False))
    def emb_kernel(table_hbm, idx_hbm, out_hbm, idx_v, rows_v, gsem, osem):
        wid = lax.axis_index("subcore") * NCORES + lax.axis_index("core")
        base = wid * per_w
        pltpu.sync_copy(idx_hbm.at[pl.ds(base, per_w)], idx_v)

        def gather(c):
            return pltpu.async_copy(
                table_hbm.at[idx_v.at[pl.ds(c * W, W)]],
                rows_v.at[c % NBUF], gsem.at[c % NBUF])

        def flush(c):
            p0 = base + c * W
            l_pos = p0 // b
            r = p0 % b
            h = r // half
            row0 = l_pos * half + r % half
            return pltpu.async_copy(
                rows_v.at[c % NBUF].at[:, pl.ds(0, D_MODEL)],
                out_hbm.at[pl.ds(row0, W), pl.ds(h * D_MODEL, D_MODEL)],
                osem.at[c % NBUF])

        ghandles = [gather(0), gather(1)]
        ohandles = [None] * NBUF
        for c in range(nchunk):
            bb = c % NBUF
            if c + 2 < nchunk:
                nb = (c + 2) % NBUF
                if ohandles[nb] is not None:
                    ohandles[nb].wait()
                ghandles.append(gather(c + 2))
            ghandles[c].wait()
            ohandles[bb] = flush(c)
        for h in ohandles:
            if h is not None:
                h.wait()

    return emb_kernel(table128, idx)


def _transpose_scale(y128, b, l):
    nj = b // 2 // B0

    def body(y_ref, o_ref):
        val = y_ref[...]
        h = pl.program_id(2)
        sel = jnp.where(h == 0, val[:, :D_MODEL], val[:, D_MODEL:])
        o_ref[...] = (sel.T * SCALE)[None]

    return pl.pallas_call(
        body,
        grid=(l, nj, 2),
        in_specs=[pl.BlockSpec((B0, 128), lambda i, jj, h: (i * nj + jj, 0))],
        out_specs=pl.BlockSpec((1, D_MODEL, B0),
                               lambda i, jj, h: (i, 0, h * nj + jj)),
        out_shape=jax.ShapeDtypeStruct((l, D_MODEL, b), jnp.float32),
    )(y128)


def kernel(x, table):
    b, l = x.shape
    n = b * l
    idx = x.T.reshape(n)
    table128 = _prep_table(table.T)
    y128 = _gather_rows(table128, idx, n, b)
    out_t = _transpose_scale(y128, b, l)
    return out_t.transpose(2, 0, 1)


# speedup vs baseline: 2.6710x; 1.3168x over previous
import jax
import jax.numpy as jnp
from jax import lax
from jax.experimental import pallas as pl
from jax.experimental.pallas import tpu as pltpu
from jax.experimental.pallas import tpu_sc as plsc

D_MODEL = 64
SCALE = 8.0
NCORES = 2
NSUB = 16
NW = NCORES * NSUB
W = 256
NBUF = 3
V0 = 4096
B0 = 2048


def _prep_table(table_t):
    d, v = table_t.shape
    grid = (v + V0 - 1) // V0

    def body(t_ref, o_ref):
        o_ref[:, :D_MODEL] = t_ref[...].T

    return pl.pallas_call(
        body,
        grid=(grid,),
        in_specs=[pl.BlockSpec((d, V0), lambda i: (0, i))],
        out_specs=pl.BlockSpec((V0, 128), lambda i: (i, 0)),
        out_shape=jax.ShapeDtypeStruct((v, 128), jnp.float32),
        compiler_params=pltpu.CompilerParams(
            dimension_semantics=("parallel",)),
    )(table_t)


def _gather_rows(table128, idx, n, b):
    per_w = n // NW
    nchunk = per_w // W
    half = b // 2

    mesh = plsc.VectorSubcoreMesh(core_axis_name="core",
                                  subcore_axis_name="subcore")

    @pl.kernel(out_type=jax.ShapeDtypeStruct((n // 2, 128), jnp.float32),
               mesh=mesh,
               scratch_types=[
                   pltpu.VMEM((per_w,), jnp.int32),
                   pltpu.VMEM((NBUF, W, 128), jnp.float32),
                   pltpu.SemaphoreType.DMA((NBUF,)),
                   pltpu.SemaphoreType.DMA((NBUF,)),
               ],
               compiler_params=pltpu.CompilerParams(use_tc_tiling_on_sc=False))
    def emb_kernel(table_hbm, idx_hbm, out_hbm, idx_v, rows_v, gsem, osem):
        wid = lax.axis_index("subcore") * NCORES + lax.axis_index("core")
        base = wid * per_w
        pltpu.sync_copy(idx_hbm.at[pl.ds(base, per_w)], idx_v)

        def gather(c):
            return pltpu.async_copy(
                table_hbm.at[idx_v.at[pl.ds(c * W, W)]],
                rows_v.at[c % NBUF], gsem.at[c % NBUF])

        def flush(c):
            p0 = base + c * W
            l_pos = p0 // b
            r = p0 % b
            h = r // half
            row0 = l_pos * half + r % half
            return pltpu.async_copy(
                rows_v.at[c % NBUF].at[:, pl.ds(0, D_MODEL)],
                out_hbm.at[pl.ds(row0, W), pl.ds(h * D_MODEL, D_MODEL)],
                osem.at[c % NBUF])

        ghandles = [gather(0), gather(1)]
        ohandles = [None] * NBUF
        for c in range(nchunk):
            bb = c % NBUF
            if c + 2 < nchunk:
                nb = (c + 2) % NBUF
                if ohandles[nb] is not None:
                    ohandles[nb].wait()
                ghandles.append(gather(c + 2))
            ghandles[c].wait()
            ohandles[bb] = flush(c)
        for h in ohandles:
            if h is not None:
                h.wait()

    return emb_kernel(table128, idx)


def _transpose_scale(y128, b, l):
    nj = b // 2 // B0

    def body(y_ref, o_ref):
        val = y_ref[...]
        h = pl.program_id(2)
        sel = jnp.where(h == 0, val[:, :D_MODEL], val[:, D_MODEL:])
        o_ref[...] = (sel.T * SCALE)[None]

    return pl.pallas_call(
        body,
        grid=(l, nj, 2),
        in_specs=[pl.BlockSpec((B0, 128), lambda i, jj, h: (i * nj + jj, 0))],
        out_specs=pl.BlockSpec((1, D_MODEL, B0),
                               lambda i, jj, h: (i, 0, h * nj + jj)),
        out_shape=jax.ShapeDtypeStruct((l, D_MODEL, b), jnp.float32),
        compiler_params=pltpu.CompilerParams(
            dimension_semantics=("parallel", "parallel", "parallel")),
    )(y128)


def kernel(x, table):
    b, l = x.shape
    n = b * l
    idx = x.T.reshape(n)
    table128 = _prep_table(table.T)
    y128 = _gather_rows(table128, idx, n, b)
    out_t = _transpose_scale(y128, b, l)
    return out_t.transpose(2, 0, 1)


# speedup vs baseline: 3.2427x; 1.2141x over previous
import jax
import jax.numpy as jnp
from jax import lax
from jax.experimental import pallas as pl
from jax.experimental.pallas import tpu as pltpu
from jax.experimental.pallas import tpu_sc as plsc

D_MODEL = 64
SCALE = 8.0
NCORES = 2
NSUB = 16
NW = NCORES * NSUB
W = 256
NBUF = 3
V0 = 8192
B0 = 4096


def _prep_table(table_t):
    d, v = table_t.shape
    grid = (v + V0 - 1) // V0

    def body(t_ref, o_ref):
        o_ref[:, :D_MODEL] = t_ref[...].T

    return pl.pallas_call(
        body,
        grid=(grid,),
        in_specs=[pl.BlockSpec((d, V0), lambda i: (0, i))],
        out_specs=pl.BlockSpec((V0, 128), lambda i: (i, 0)),
        out_shape=jax.ShapeDtypeStruct((v, 128), jnp.float32),
        compiler_params=pltpu.CompilerParams(
            dimension_semantics=("parallel",)),
    )(table_t)


def _gather_rows(table128, idx, n, b):
    per_w = n // NW
    nchunk = per_w // W
    half = b // 2

    mesh = plsc.VectorSubcoreMesh(core_axis_name="core",
                                  subcore_axis_name="subcore")

    @pl.kernel(out_type=jax.ShapeDtypeStruct((n // 2, 128), jnp.float32),
               mesh=mesh,
               scratch_types=[
                   pltpu.VMEM((per_w,), jnp.int32),
                   pltpu.VMEM((NBUF, W, 128), jnp.float32),
                   pltpu.SemaphoreType.DMA((NBUF,)),
                   pltpu.SemaphoreType.DMA((NBUF,)),
               ],
               compiler_params=pltpu.CompilerParams(use_tc_tiling_on_sc=False))
    def emb_kernel(table_hbm, idx_hbm, out_hbm, idx_v, rows_v, gsem, osem):
        wid = lax.axis_index("subcore") * NCORES + lax.axis_index("core")
        base = wid * per_w
        pltpu.sync_copy(idx_hbm.at[pl.ds(base, per_w)], idx_v)

        def gather(c):
            return pltpu.async_copy(
                table_hbm.at[idx_v.at[pl.ds(c * W, W)]],
                rows_v.at[c % NBUF], gsem.at[c % NBUF])

        def flush(c):
            p0 = base + c * W
            l_pos = p0 // b
            r = p0 % b
            h = r // half
            row0 = l_pos * half + r % half
            return pltpu.async_copy(
                rows_v.at[c % NBUF].at[:, pl.ds(0, D_MODEL)],
                out_hbm.at[pl.ds(row0, W), pl.ds(h * D_MODEL, D_MODEL)],
                osem.at[c % NBUF])

        ghandles = [gather(0), gather(1)]
        ohandles = [None] * NBUF
        for c in range(nchunk):
            bb = c % NBUF
            if c + 2 < nchunk:
                nb = (c + 2) % NBUF
                if ohandles[nb] is not None:
                    ohandles[nb].wait()
                ghandles.append(gather(c + 2))
            ghandles[c].wait()
            ohandles[bb] = flush(c)
        for h in ohandles:
            if h is not None:
                h.wait()

    return emb_kernel(table128, idx)


def _transpose_scale(y128, b, l):
    nj = b // 2 // B0

    def body(y_ref, o_ref):
        val = y_ref[...]
        h = pl.program_id(2)
        sel = jnp.where(h == 0, val[:, :D_MODEL], val[:, D_MODEL:])
        o_ref[...] = (sel.T * SCALE)[None]

    return pl.pallas_call(
        body,
        grid=(l, nj, 2),
        in_specs=[pl.BlockSpec((B0, 128), lambda i, jj, h: (i * nj + jj, 0))],
        out_specs=pl.BlockSpec((1, D_MODEL, B0),
                               lambda i, jj, h: (i, 0, h * nj + jj)),
        out_shape=jax.ShapeDtypeStruct((l, D_MODEL, b), jnp.float32),
        compiler_params=pltpu.CompilerParams(
            dimension_semantics=("parallel", "parallel", "parallel")),
    )(y128)


def kernel(x, table):
    b, l = x.shape
    n = b * l
    idx = x.T.reshape(n)
    table128 = _prep_table(table.T)
    y128 = _gather_rows(table128, idx, n, b)
    out_t = _transpose_scale(y128, b, l)
    return out_t.transpose(2, 0, 1)


# speedup vs baseline: 3.5756x; 1.1026x over previous
import jax
import jax.numpy as jnp
from jax import lax
from jax.experimental import pallas as pl
from jax.experimental.pallas import tpu as pltpu
from jax.experimental.pallas import tpu_sc as plsc

D_MODEL = 64
SCALE = 8.0
NCORES = 2
NSUB = 16
NW = NCORES * NSUB
W = 256
NBUF = 3
V0 = 16384
B0 = 8192


def _prep_table(table_t):
    d, v = table_t.shape
    grid = (v + V0 - 1) // V0

    def body(t_ref, o_ref):
        o_ref[:, :D_MODEL] = t_ref[...].T

    return pl.pallas_call(
        body,
        grid=(grid,),
        in_specs=[pl.BlockSpec((d, V0), lambda i: (0, i))],
        out_specs=pl.BlockSpec((V0, 128), lambda i: (i, 0)),
        out_shape=jax.ShapeDtypeStruct((v, 128), jnp.float32),
        compiler_params=pltpu.CompilerParams(
            dimension_semantics=("parallel",)),
    )(table_t)


def _gather_rows(table128, idx, n, b):
    per_w = n // NW
    nchunk = per_w // W
    half = b // 2

    mesh = plsc.VectorSubcoreMesh(core_axis_name="core",
                                  subcore_axis_name="subcore")

    @pl.kernel(out_type=jax.ShapeDtypeStruct((n // 2, 128), jnp.float32),
               mesh=mesh,
               scratch_types=[
                   pltpu.VMEM((per_w,), jnp.int32),
                   pltpu.VMEM((NBUF, W, 128), jnp.float32),
                   pltpu.SemaphoreType.DMA((NBUF,)),
                   pltpu.SemaphoreType.DMA((NBUF,)),
               ],
               compiler_params=pltpu.CompilerParams(use_tc_tiling_on_sc=False))
    def emb_kernel(table_hbm, idx_hbm, out_hbm, idx_v, rows_v, gsem, osem):
        wid = lax.axis_index("subcore") * NCORES + lax.axis_index("core")
        base = wid * per_w
        pltpu.sync_copy(idx_hbm.at[pl.ds(base, per_w)], idx_v)

        def gather(c):
            return pltpu.async_copy(
                table_hbm.at[idx_v.at[pl.ds(c * W, W)]],
                rows_v.at[c % NBUF], gsem.at[c % NBUF])

        def flush(c):
            p0 = base + c * W
            l_pos = p0 // b
            r = p0 % b
            h = r // half
            row0 = l_pos * half + r % half
            return pltpu.async_copy(
                rows_v.at[c % NBUF].at[:, pl.ds(0, D_MODEL)],
                out_hbm.at[pl.ds(row0, W), pl.ds(h * D_MODEL, D_MODEL)],
                osem.at[c % NBUF])

        ghandles = [gather(0), gather(1)]
        ohandles = [None] * NBUF
        for c in range(nchunk):
            bb = c % NBUF
            if c + 2 < nchunk:
                nb = (c + 2) % NBUF
                if ohandles[nb] is not None:
                    ohandles[nb].wait()
                ghandles.append(gather(c + 2))
            ghandles[c].wait()
            ohandles[bb] = flush(c)
        for h in ohandles:
            if h is not None:
                h.wait()

    return emb_kernel(table128, idx)


def _transpose_scale(y128, b, l):
    nj = b // 2 // B0

    def body(y_ref, o_ref):
        val = y_ref[...]
        h = pl.program_id(2)
        sel = jnp.where(h == 0, val[:, :D_MODEL], val[:, D_MODEL:])
        o_ref[...] = (sel.T * SCALE)[None]

    return pl.pallas_call(
        body,
        grid=(l, nj, 2),
        in_specs=[pl.BlockSpec((B0, 128), lambda i, jj, h: (i * nj + jj, 0))],
        out_specs=pl.BlockSpec((1, D_MODEL, B0),
                               lambda i, jj, h: (i, 0, h * nj + jj)),
        out_shape=jax.ShapeDtypeStruct((l, D_MODEL, b), jnp.float32),
        compiler_params=pltpu.CompilerParams(
            dimension_semantics=("parallel", "parallel", "parallel")),
    )(y128)


def kernel(x, table):
    b, l = x.shape
    n = b * l
    idx = x.T.reshape(n)
    table128 = _prep_table(table.T)
    y128 = _gather_rows(table128, idx, n, b)
    out_t = _transpose_scale(y128, b, l)
    return out_t.transpose(2, 0, 1)


# speedup vs baseline: 3.6104x; 1.0098x over previous
import jax
import jax.numpy as jnp
from jax import lax
from jax.experimental import pallas as pl
from jax.experimental.pallas import tpu as pltpu
from jax.experimental.pallas import tpu_sc as plsc

D_MODEL = 64
SCALE = 8.0
NCORES = 2
NSUB = 16
NW = NCORES * NSUB
W = 256
NBUF = 3
V0 = 32768
B0 = 8192


def _prep_table(table_t):
    d, v = table_t.shape
    grid = (v + V0 - 1) // V0

    def body(t_ref, o_ref):
        o_ref[:, :D_MODEL] = t_ref[...].T

    return pl.pallas_call(
        body,
        grid=(grid,),
        in_specs=[pl.BlockSpec((d, V0), lambda i: (0, i))],
        out_specs=pl.BlockSpec((V0, 128), lambda i: (i, 0)),
        out_shape=jax.ShapeDtypeStruct((v, 128), jnp.float32),
        compiler_params=pltpu.CompilerParams(
            dimension_semantics=("parallel",)),
    )(table_t)


def _gather_rows(table128, idx, n, b):
    per_w = n // NW
    nchunk = per_w // W
    half = b // 2

    mesh = plsc.VectorSubcoreMesh(core_axis_name="core",
                                  subcore_axis_name="subcore")

    @pl.kernel(out_type=jax.ShapeDtypeStruct((n // 2, 128), jnp.float32),
               mesh=mesh,
               scratch_types=[
                   pltpu.VMEM((per_w,), jnp.int32),
                   pltpu.VMEM((NBUF, W, 128), jnp.float32),
                   pltpu.SemaphoreType.DMA((NBUF,)),
                   pltpu.SemaphoreType.DMA((NBUF,)),
               ],
               compiler_params=pltpu.CompilerParams(use_tc_tiling_on_sc=False))
    def emb_kernel(table_hbm, idx_hbm, out_hbm, idx_v, rows_v, gsem, osem):
        wid = lax.axis_index("subcore") * NCORES + lax.axis_index("core")
        base = wid * per_w
        pltpu.sync_copy(idx_hbm.at[pl.ds(base, per_w)], idx_v)

        def gather(c):
            return pltpu.async_copy(
                table_hbm.at[idx_v.at[pl.ds(c * W, W)]],
                rows_v.at[c % NBUF], gsem.at[c % NBUF])

        def flush(c):
            p0 = base + c * W
            l_pos = p0 // b
            r = p0 % b
            h = r // half
            row0 = l_pos * half + r % half
            return pltpu.async_copy(
                rows_v.at[c % NBUF].at[:, pl.ds(0, D_MODEL)],
                out_hbm.at[pl.ds(row0, W), pl.ds(h * D_MODEL, D_MODEL)],
                osem.at[c % NBUF])

        ghandles = [gather(0), gather(1)]
        ohandles = [None] * NBUF
        for c in range(nchunk):
            bb = c % NBUF
            if c + 2 < nchunk:
                nb = (c + 2) % NBUF
                if ohandles[nb] is not None:
                    ohandles[nb].wait()
                ghandles.append(gather(c + 2))
            ghandles[c].wait()
            ohandles[bb] = flush(c)
        for h in ohandles:
            if h is not None:
                h.wait()

    return emb_kernel(table128, idx)


def _transpose_scale(y128, b, l):
    nj = b // 2 // B0

    def body(y_ref, o_ref):
        val = y_ref[...]
        h = pl.program_id(2)
        sel = jnp.where(h == 0, val[:, :D_MODEL], val[:, D_MODEL:])
        o_ref[...] = (sel.T * SCALE)[None]

    return pl.pallas_call(
        body,
        grid=(l, nj, 2),
        in_specs=[pl.BlockSpec((B0, 128), lambda i, jj, h: (i * nj + jj, 0))],
        out_specs=pl.BlockSpec((1, D_MODEL, B0),
                               lambda i, jj, h: (i, 0, h * nj + jj)),
        out_shape=jax.ShapeDtypeStruct((l, D_MODEL, b), jnp.float32),
        compiler_params=pltpu.CompilerParams(
            dimension_semantics=("parallel", "parallel", "parallel")),
    )(y128)


def kernel(x, table):
    b, l = x.shape
    n = b * l
    idx = x.T.reshape(n)
    table128 = _prep_table(table.T)
    y128 = _gather_rows(table128, idx, n, b)
    out_t = _transpose_scale(y128, b, l)
    return out_t.transpose(2, 0, 1)


# speedup vs baseline: 3.8205x; 1.0582x over previous
import jax
import jax.numpy as jnp
from jax import lax
from jax.experimental import pallas as pl
from jax.experimental.pallas import tpu as pltpu
from jax.experimental.pallas import tpu_sc as plsc

D_MODEL = 64
SCALE = 8.0
NCORES = 2
NSUB = 16
NW = NCORES * NSUB
W = 256
NBUF = 3
V0 = 32768
B0 = 8192


def _prep_table(table_t):
    d, v = table_t.shape
    grid = (v + V0 - 1) // V0

    def body(t_ref, o_ref):
        o_ref[:, :D_MODEL] = t_ref[...].T

    return pl.pallas_call(
        body,
        grid=(grid,),
        in_specs=[pl.BlockSpec((d, V0), lambda i: (0, i))],
        out_specs=pl.BlockSpec((V0, 128), lambda i: (i, 0)),
        out_shape=jax.ShapeDtypeStruct((v, 128), jnp.float32),
        compiler_params=pltpu.CompilerParams(
            dimension_semantics=("parallel",)),
    )(table_t)


def _gather_rows(table128, idx, n, b):
    per_w = n // NW
    nchunk = per_w // W
    half = b // 2

    mesh = plsc.VectorSubcoreMesh(core_axis_name="core",
                                  subcore_axis_name="subcore")

    @pl.kernel(out_type=jax.ShapeDtypeStruct((n // 2, 128), jnp.float32),
               mesh=mesh,
               scratch_types=[
                   pltpu.VMEM((per_w,), jnp.int32),
                   pltpu.VMEM((NBUF, W, 128), jnp.float32),
                   pltpu.SemaphoreType.DMA((NBUF,)),
                   pltpu.SemaphoreType.DMA((NBUF,)),
               ],
               compiler_params=pltpu.CompilerParams(use_tc_tiling_on_sc=False))
    def emb_kernel(table_hbm, idx_hbm, out_hbm, idx_v, rows_v, gsem, osem):
        wid = lax.axis_index("subcore") * NCORES + lax.axis_index("core")
        base = wid * per_w
        pltpu.sync_copy(idx_hbm.at[pl.ds(base, per_w)], idx_v)

        def gather(c):
            return pltpu.async_copy(
                table_hbm.at[idx_v.at[pl.ds(c * W, W)]],
                rows_v.at[c % NBUF], gsem.at[c % NBUF])

        def flush(c):
            p0 = base + c * W
            l_pos = p0 // b
            r = p0 % b
            h = r // half
            row0 = l_pos * half + r % half
            return pltpu.async_copy(
                rows_v.at[c % NBUF].at[:, pl.ds(0, D_MODEL)],
                out_hbm.at[pl.ds(row0, W), pl.ds(h * D_MODEL, D_MODEL)],
                osem.at[c % NBUF])

        ghandles = [gather(0), gather(1)]
        ohandles = [None] * NBUF
        for c in range(nchunk):
            bb = c % NBUF
            if c + 2 < nchunk:
                nb = (c + 2) % NBUF
                if ohandles[nb] is not None:
                    ohandles[nb].wait()
                ghandles.append(gather(c + 2))
            ghandles[c].wait()
            ohandles[bb] = flush(c)
        for h in ohandles:
            if h is not None:
                h.wait()

    return emb_kernel(table128, idx)


def _transpose_scale(y128, b, l, lhalf, loff, out_prev=None):
    nj = b // 2 // B0

    def body(y_ref, *refs):
        o_ref = refs[-1]
        val = y_ref[...]
        h = pl.program_id(2)
        sel = jnp.where(h == 0, val[:, :D_MODEL], val[:, D_MODEL:])
        o_ref[...] = (sel.T * SCALE)[None]

    operands = [y128]
    in_specs = [pl.BlockSpec((B0, 128), lambda i, jj, h: (i * nj + jj, 0))]
    aliases = {}
    if out_prev is not None:
        operands.append(out_prev)
        in_specs.append(pl.BlockSpec(memory_space=pl.ANY))
        aliases = {1: 0}

    return pl.pallas_call(
        body,
        grid=(lhalf, nj, 2),
        in_specs=in_specs,
        out_specs=pl.BlockSpec((1, D_MODEL, B0),
                               lambda i, jj, h: (i + loff, 0, h * nj + jj)),
        out_shape=jax.ShapeDtypeStruct((l, D_MODEL, b), jnp.float32),
        input_output_aliases=aliases,
        compiler_params=pltpu.CompilerParams(
            dimension_semantics=("parallel", "parallel", "parallel")),
    )(*operands)


def kernel(x, table):
    b, l = x.shape
    n = b * l
    idx = x.T.reshape(n)
    table128 = _prep_table(table.T)
    lh = l // 2
    nh = n // 2
    y_a = _gather_rows(table128, idx[:nh], nh, b)
    y_b = _gather_rows(table128, idx[nh:], nh, b)
    out_a = _transpose_scale(y_a, b, l, lh, 0)
    out_t = _transpose_scale(y_b, b, l, lh, lh, out_prev=out_a)
    return out_t.transpose(2, 0, 1)
